# baseline (speedup 1.0000x reference)
import functools

import jax
import jax.numpy as jnp
from jax import lax
from jax.experimental import pallas as pl
from jax.experimental.pallas import tpu as pltpu
from jax.experimental.pallas import tpu_sc as plsc

_ROWS = 128
_N = 32768
_K = 328
_L = 16
_NC = 2
_NS = 16
_NW = _NC * _NS
_ROWS_PER_W = _ROWS // _NW
_NCHUNK = _N // _L
_NSLOT = 7
_UNROLL = 7
_HWORDS = 256 * _L

_mesh = plsc.VectorSubcoreMesh(core_axis_name="c", subcore_axis_name="s")


@functools.partial(
    pl.kernel,
    mesh=_mesh,
    out_type=jax.ShapeDtypeStruct((_ROWS, _N), jnp.float32),
    compiler_params=pltpu.CompilerParams(needs_layout_passes=False),
    scratch_types=[
        pltpu.VMEM((_L,), jnp.float32),
        pltpu.VMEM((_N,), jnp.float32),
        pltpu.VMEM((_N,), jnp.float32),
        pltpu.VMEM((_N,), jnp.int32),
        pltpu.VMEM((_NSLOT * _HWORDS,), jnp.int32),
        pltpu.SMEM((256,), jnp.int32),
    ],
)
def _sc_kwinner(
    td_hbm, duty_hbm, x_hbm, out_hbm, td_v, duty_v, x_v, v_v, hist_v, tot_s
):
    wid = lax.axis_index("s") * _NC + lax.axis_index("c")
    pltpu.sync_copy(td_hbm, td_v)
    pltpu.sync_copy(duty_hbm, duty_v)
    td = td_v[...]
    lane = lax.iota(jnp.int32, _L)
    ones = jnp.full((_L,), 1, jnp.int32)
    zeros16 = jnp.zeros((_L,), jnp.int32)

    @plsc.parallel_loop(0, _NSLOT * 256, unroll=8)
    def _zero(j):
        hist_v[pl.ds(j * _L, _L)] = zeros16

    @plsc.parallel_loop(0, _NCHUNK, unroll=8)
    def _bf(i):
        sl = pl.ds(i * _L, _L)
        duty_v[sl] = jnp.exp(td - duty_v[sl])

    def scan_bins(rank):
        @plsc.parallel_loop(
            0, 256, unroll=16, carry=jnp.zeros((_L,), jnp.int32)
        )
        def sup(j, sup_acc):
            sls = [pl.ds(s * _HWORDS + j * _L, _L) for s in range(_NSLOT)]
            acc = hist_v[sls[0]]
            for sl in sls[1:]:
                acc = acc + hist_v[sl]
            t = jnp.sum(acc)
            tot_s[j] = t
            for sl in sls:
                hist_v[sl] = zeros16
            tsplat = jnp.full((_L,), t, jnp.int32)
            sel = lane == lax.shift_right_logical(j, 4)
            return sup_acc + jnp.where(sel, tsplat, 0)

        rev = lax.rev(sup, (0,))
        cs = plsc.cumsum(rev)
        ge = cs >= jnp.full((_L,), rank, jnp.int32)
        pc = plsc.all_reduce_population_count(ge)
        super_bin = jnp.max(pc) - 1
        isel = lane == (jnp.full((_L,), 16, jnp.int32) - pc)
        above = jnp.sum(jnp.where(isel, cs - rev, 0))
        rem0 = rank - above

        def sb(jj, carry):
            found, rem = carry
            j = super_bin * 16 + (jnp.int32(15) - jj)
            cnt = tot_s[j]
            miss = found < 0
            hit = miss & (rem <= cnt)
            found = jnp.where(hit, j, found)
            rem = jnp.where(miss & jnp.logical_not(hit), rem - cnt, rem)
            return found, rem

        return lax.fori_loop(0, 16, sb, (jnp.int32(-1), rem0))

    def row_body(r, c0):
        row = wid * _ROWS_PER_W + r
        pltpu.sync_copy(x_hbm.at[row], x_v)

        @plsc.parallel_loop(0, _NCHUNK, unroll=_UNROLL)
        def _p1(i):
            sl = pl.ds(i * _L, _L)
            xv = x_v[sl]
            b = xv * duty_v[sl] + 0.0
            s = lax.bitcast_convert_type(b, jnp.int32)
            v = s ^ (lax.shift_right_arithmetic(s, 31) & jnp.int32(0x7FFFFFFF))
            v_v[sl] = v
            digit = lax.shift_right_logical(v, 24) ^ 128
            hidx = (lax.rem(i, _NSLOT) * _HWORDS) + ((digit << 4) | lane)
            plsc.addupdate_scatter(hist_v, [hidx], ones)

        d1, rank = scan_bins(jnp.int32(_K))
        t_hi = (d1 ^ 128) << 24

        def radix_pass(shift, t_hi, rank):
            pshift = shift + 8
            tp = lax.shift_right_logical(t_hi, pshift)

            @plsc.parallel_loop(0, _NCHUNK, unroll=_UNROLL)
            def _pp(i):
                v = v_v[pl.ds(i * _L, _L)]
                match = lax.shift_right_logical(v, pshift) == tp
                digit = lax.shift_right_logical(v, shift) & 255
                hidx = (lax.rem(i, _NSLOT) * _HWORDS) + ((digit << 4) | lane)
                plsc.addupdate_scatter(hist_v, [hidx], ones, mask=match)

            d, rank2 = scan_bins(rank)
            return t_hi | (d << shift), rank2

        t_hi, rank = radix_pass(16, t_hi, rank)
        t_hi, rank = radix_pass(8, t_hi, rank)
        t_hi, rank = radix_pass(0, t_hi, rank)
        tv = jnp.full((_L,), t_hi, jnp.int32)

        @plsc.parallel_loop(0, _NCHUNK, unroll=16)
        def _pmask(i):
            sl = pl.ds(i * _L, _L)
            x_v[sl] = jnp.where(v_v[sl] >= tv, x_v[sl], jnp.float32(0.0))

        pltpu.sync_copy(x_v, out_hbm.at[row])
        return c0

    lax.fori_loop(0, _ROWS_PER_W, row_body, jnp.int32(0))


def kernel(x, dutyCycles, k):
    td = jnp.full((_L,), jnp.float32(k) / jnp.float32(x.shape[1]), jnp.float32)
    return _sc_kwinner(td, dutyCycles, x)


# speedup vs baseline: 1.1048x; 1.1048x over previous
import functools

import jax
import jax.numpy as jnp
from jax import lax
from jax.experimental import pallas as pl
from jax.experimental.pallas import tpu as pltpu
from jax.experimental.pallas import tpu_sc as plsc

_ROWS = 128
_N = 32768
_K = 328
_L = 16
_NC = 2
_NS = 16
_NW = _NC * _NS
_ROWS_PER_W = _ROWS // _NW
_NCHUNK = _N // _L
_NSLOT = 7
_UNROLL = 7
_HWORDS = 256 * _L

_mesh = plsc.VectorSubcoreMesh(core_axis_name="c", subcore_axis_name="s")


@functools.partial(
    pl.kernel,
    mesh=_mesh,
    out_type=jax.ShapeDtypeStruct((_ROWS, _N), jnp.float32),
    compiler_params=pltpu.CompilerParams(needs_layout_passes=False),
    scratch_types=[
        pltpu.VMEM((_L,), jnp.float32),
        pltpu.VMEM((_N,), jnp.float32),
        pltpu.VMEM((_N,), jnp.float32),
        pltpu.VMEM((_N,), jnp.int32),
        pltpu.VMEM((_NSLOT * _HWORDS,), jnp.int32),
        pltpu.SMEM((256,), jnp.int32),
    ],
)
def _sc_kwinner(
    td_hbm, duty_hbm, x_hbm, out_hbm, td_v, duty_v, x_v, v_v, hist_v, tot_s
):
    wid = lax.axis_index("s") * _NC + lax.axis_index("c")
    pltpu.sync_copy(td_hbm, td_v)
    pltpu.sync_copy(duty_hbm, duty_v)
    td = td_v[...]
    lane = lax.iota(jnp.int32, _L)
    ones = jnp.full((_L,), 1, jnp.int32)
    zeros16 = jnp.zeros((_L,), jnp.int32)

    @plsc.parallel_loop(0, _NSLOT * 256, unroll=8)
    def _zero(j):
        hist_v[pl.ds(j * _L, _L)] = zeros16

    @plsc.parallel_loop(0, _NCHUNK, unroll=8)
    def _bf(i):
        sl = pl.ds(i * _L, _L)
        duty_v[sl] = jnp.exp(td - duty_v[sl])

    def scan_bins(rank):
        @plsc.parallel_loop(
            0, 256, unroll=8, carry=jnp.zeros((_L,), jnp.int32)
        )
        def sup(j, sup_acc):
            sls = [pl.ds(s * _HWORDS + j * _L, _L) for s in range(_NSLOT)]
            acc = hist_v[sls[0]]
            for sl in sls[1:]:
                acc = acc + hist_v[sl]
            t = jnp.sum(acc)
            tot_s[j] = t
            for sl in sls:
                hist_v[sl] = zeros16
            tsplat = jnp.full((_L,), t, jnp.int32)
            sel = lane == lax.shift_right_logical(j, 4)
            return sup_acc + jnp.where(sel, tsplat, 0)

        rev = lax.rev(sup, (0,))
        cs = plsc.cumsum(rev)
        ge = cs >= jnp.full((_L,), rank, jnp.int32)
        pc = plsc.all_reduce_population_count(ge)
        super_bin = jnp.max(pc) - 1
        isel = lane == (jnp.full((_L,), 16, jnp.int32) - pc)
        above = jnp.sum(jnp.where(isel, cs - rev, 0))
        rem0 = rank - above

        def sb(jj, carry):
            found, rem = carry
            j = super_bin * 16 + (jnp.int32(15) - jj)
            cnt = tot_s[j]
            miss = found < 0
            hit = miss & (rem <= cnt)
            found = jnp.where(hit, j, found)
            rem = jnp.where(miss & jnp.logical_not(hit), rem - cnt, rem)
            return found, rem

        return lax.fori_loop(0, 16, sb, (jnp.int32(-1), rem0))

    def row_body(r, c0):
        row = wid * _ROWS_PER_W + r
        pltpu.sync_copy(x_hbm.at[row], x_v)

        @plsc.parallel_loop(0, _NCHUNK, unroll=_UNROLL)
        def _p1(i):
            sl = pl.ds(i * _L, _L)
            xv = x_v[sl]
            b = xv * duty_v[sl] + 0.0
            s = lax.bitcast_convert_type(b, jnp.int32)
            v = s ^ (lax.shift_right_arithmetic(s, 31) & jnp.int32(0x7FFFFFFF))
            v_v[sl] = v
            digit = lax.shift_right_logical(v, 24) ^ 128
            hidx = (lax.rem(i, _NSLOT) * _HWORDS) + ((digit << 4) | lane)
            plsc.addupdate_scatter(hist_v, [hidx], ones)

        d1, rank = scan_bins(jnp.int32(_K))
        t_hi = (d1 ^ 128) << 24

        def radix_pass(shift, t_hi, rank):
            pshift = shift + 8
            tp = lax.shift_right_logical(t_hi, pshift)

            @plsc.parallel_loop(0, _NCHUNK, unroll=_UNROLL)
            def _pp(i):
                v = v_v[pl.ds(i * _L, _L)]
                match = lax.shift_right_logical(v, pshift) == tp
                digit = lax.shift_right_logical(v, shift) & 255
                hidx = (lax.rem(i, _NSLOT) * _HWORDS) + ((digit << 4) | lane)
                plsc.addupdate_scatter(hist_v, [hidx], ones, mask=match)

            d, rank2 = scan_bins(rank)
            return t_hi | (d << shift), rank2

        t_hi, rank = radix_pass(16, t_hi, rank)
        t_hi, rank = radix_pass(8, t_hi, rank)
        t_hi, rank = radix_pass(0, t_hi, rank)
        tv = jnp.full((_L,), t_hi, jnp.int32)

        @plsc.parallel_loop(0, _NCHUNK, unroll=16)
        def _pmask(i):
            sl = pl.ds(i * _L, _L)
            x_v[sl] = jnp.where(v_v[sl] >= tv, x_v[sl], jnp.float32(0.0))

        pltpu.sync_copy(x_v, out_hbm.at[row])
        return c0

    lax.fori_loop(0, _ROWS_PER_W, row_body, jnp.int32(0))


def kernel(x, dutyCycles, k):
    td = jnp.full((_L,), jnp.float32(k) / jnp.float32(x.shape[1]), jnp.float32)
    return _sc_kwinner(td, dutyCycles, x)
